# baseline (speedup 1.0000x reference)
import jax
import jax.numpy as jnp
from jax import lax
from jax.experimental import pallas as pl
from jax.experimental.pallas import tpu as pltpu
from jax.experimental.pallas import tpu_sc as plsc

B = 1024
FAN0 = 25
FAN1 = 10
D = 128
HID = 128

NC = 2
NS = 16
NW = NC * NS

M_ROWS = B + B * FAN1
MR_PER_W = M_ROWS // NW
M_CHUNK = 8
M_CHUNKS = MR_PER_W // M_CHUNK
N1_CHUNKS = B // M_CHUNK

H_ROWS = B * FAN1
HR_PER_W = H_ROWS // NW
H_GLEN = 64
H_GATHERS = HR_PER_W // H_GLEN


def _mean_pipeline(feats_hbm, idx_hbm, out_ref, idxv, buf, obuf, msem,
                   chunk0, n_chunks, out_row0):
    def fire_m(c, b):
        pltpu.sync_copy(idx_hbm.at[pl.ds((chunk0 + c) * M_CHUNK, M_CHUNK)],
                        idxv[b])
        for r in range(M_CHUNK):
            pltpu.async_copy(feats_hbm.at[idxv[b].at[r]], buf[b].at[r],
                             msem[b])

    def drain_m(b):
        for r in range(M_CHUNK):
            pltpu.make_async_copy(feats_hbm.at[idxv[b].at[r]], buf[b].at[r],
                                  msem[b]).wait()

    def reduce_chunk(b):
        for r in range(M_CHUNK):
            accs = [buf[b][r, 0, pl.ds(16 * j, 16)] for j in range(D // 16)]

            def acc_body(k, a):
                return tuple(a[j] + buf[b][r, k, pl.ds(16 * j, 16)]
                             for j in range(D // 16))

            accs = lax.fori_loop(1, FAN0, acc_body, tuple(accs))
            for j in range(D // 16):
                obuf[r, pl.ds(16 * j, 16)] = accs[j] * (1.0 / FAN0)

    fire_m(0, 0)
    fire_m(1, 1)

    def pair_body(g, carry):
        for b in range(2):
            c = 2 * g + b
            drain_m(b)
            reduce_chunk(b)
            pltpu.sync_copy(obuf,
                            out_ref.at[pl.ds(out_row0 + c * M_CHUNK, M_CHUNK)])

            @pl.when(c + 2 < n_chunks)
            def _():
                fire_m(c + 2, b)
        return carry

    lax.fori_loop(0, n_chunks // 2, pair_body, 0)


def _sc_a_body(feats_hbm, n1_hbm, m1_out,
               idxv0, idxv1, buf0, buf1, obuf, msem0, msem1):
    wid = lax.axis_index("s") * NC + lax.axis_index("c")
    n1_chunks = N1_CHUNKS // NW
    _mean_pipeline(feats_hbm, n1_hbm, m1_out, (idxv0, idxv1), (buf0, buf1),
                   obuf, (msem0, msem1),
                   chunk0=wid * n1_chunks, n_chunks=n1_chunks,
                   out_row0=wid * (B // NW))


def _sc_b_body(feats_hbm, n2n_hbm, idxh_hbm, m2_out, hp2_out,
               idxv0, idxv1, buf0, buf1, obuf, idxh, bufh, msem0, msem1, hsem):
    wid = lax.axis_index("s") * NC + lax.axis_index("c")
    h_row0 = wid * HR_PER_W

    pltpu.sync_copy(idxh_hbm.at[pl.ds(h_row0, HR_PER_W)], idxh)
    for j in range(H_GATHERS):
        pltpu.async_copy(feats_hbm.at[idxh.at[pl.ds(j * H_GLEN, H_GLEN)]],
                         bufh.at[pl.ds(j * H_GLEN, H_GLEN)], hsem)

    n2_chunks = (H_ROWS // M_CHUNK) // NW
    _mean_pipeline(feats_hbm, n2n_hbm, m2_out, (idxv0, idxv1), (buf0, buf1),
                   obuf, (msem0, msem1),
                   chunk0=wid * n2_chunks, n_chunks=n2_chunks,
                   out_row0=wid * (H_ROWS // NW))

    for j in range(H_GATHERS):
        pltpu.make_async_copy(feats_hbm.at[idxh.at[pl.ds(j * H_GLEN, H_GLEN)]],
                              bufh.at[pl.ds(j * H_GLEN, H_GLEN)], hsem).wait()
    pltpu.sync_copy(bufh, hp2_out.at[pl.ds(h_row0, HR_PER_W)])


_MEAN_SCRATCH = [
    pltpu.VMEM((M_CHUNK, FAN0), jnp.int32),
    pltpu.VMEM((M_CHUNK, FAN0), jnp.int32),
    pltpu.VMEM((M_CHUNK, FAN0, D), jnp.float32),
    pltpu.VMEM((M_CHUNK, FAN0, D), jnp.float32),
    pltpu.VMEM((M_CHUNK, D), jnp.float32),
]


@jax.jit
def _sc_gather(feats, n1, n2n, idx_h):
    mesh = plsc.VectorSubcoreMesh(core_axis_name="c", subcore_axis_name="s",
                                  num_cores=NC, num_subcores=NS)
    m1 = pl.kernel(
        _sc_a_body,
        out_type=jax.ShapeDtypeStruct((B, D), jnp.float32),
        mesh=mesh,
        scratch_types=_MEAN_SCRATCH + [
            pltpu.SemaphoreType.DMA,
            pltpu.SemaphoreType.DMA,
        ],
    )(feats, n1)
    m2, hp2 = pl.kernel(
        _sc_b_body,
        out_type=(
            jax.ShapeDtypeStruct((B * FAN1, D), jnp.float32),
            jax.ShapeDtypeStruct((H_ROWS, D), jnp.float32),
        ),
        mesh=mesh,
        scratch_types=_MEAN_SCRATCH + [
            pltpu.VMEM((HR_PER_W,), jnp.int32),
            pltpu.VMEM((HR_PER_W, D), jnp.float32),
            pltpu.SemaphoreType.DMA,
            pltpu.SemaphoreType.DMA,
            pltpu.SemaphoreType.DMA,
        ],
    )(feats, n2n, idx_h)
    return m1, m2, hp2


def _tc_dense_body(x_ref, m1_ref, hp2_ref, m2_ref, w0_ref, b0_ref,
                   w1_ref, b1_ref, out_ref):
    def lin(a, b, w_ref, bias):
        return jnp.maximum(
            jnp.dot(a, w_ref[:D], preferred_element_type=jnp.float32)
            + jnp.dot(b, w_ref[D:], preferred_element_type=jnp.float32)
            + bias, 0.0)

    h0 = lin(x_ref[...], m1_ref[...], w0_ref, b0_ref[...])
    h2 = lin(hp2_ref[...], m2_ref[...], w0_ref, b0_ref[...])
    na = jnp.mean(h2.reshape(B, FAN1, HID), axis=1)
    out_ref[...] = lin(h0, na, w1_ref, b1_ref[...])


@jax.jit
def _tc_dense(x, m1, hp2, m2, W0, b0, W1, b1):
    return pl.pallas_call(
        _tc_dense_body,
        out_shape=jax.ShapeDtypeStruct((B, HID), jnp.float32),
    )(x, m1, hp2, m2, W0, b0.reshape(1, HID), W1, b1.reshape(1, HID))


def kernel(x, nodes, feats, neighs1, neighs2, neighs2_neighs, W0, b0, W1, b1):
    idx_h = neighs2.reshape(-1).astype(jnp.int32)
    m1, m2, hp2 = _sc_gather(feats, neighs1.astype(jnp.int32),
                             neighs2_neighs.astype(jnp.int32), idx_h)
    out = _tc_dense(x, m1, hp2, m2, W0, b0, W1, b1)
    return out.reshape(B, 1, HID)


# speedup vs baseline: 1.0068x; 1.0068x over previous
import jax
import jax.numpy as jnp
from jax import lax
from jax.experimental import pallas as pl
from jax.experimental.pallas import tpu as pltpu
from jax.experimental.pallas import tpu_sc as plsc

B = 1024
FAN0 = 25
FAN1 = 10
D = 128
HID = 128

NC = 2
NS = 16
NW = NC * NS

M_ROWS = B + B * FAN1
MR_PER_W = M_ROWS // NW
M_CHUNK = 8
M_CHUNKS = MR_PER_W // M_CHUNK
N1_CHUNKS = B // M_CHUNK

H_ROWS = B * FAN1
HR_PER_W = H_ROWS // NW
H_GLEN = 64
H_GATHERS = HR_PER_W // H_GLEN


def _mean_pipeline(feats_hbm, idx_hbm, out_ref, idxv, buf, obuf, msem,
                   chunk0, n_chunks, out_row0):
    def fire_m(c, b):
        pltpu.sync_copy(idx_hbm.at[pl.ds((chunk0 + c) * M_CHUNK, M_CHUNK)],
                        idxv[b])
        for r in range(M_CHUNK):
            pltpu.async_copy(feats_hbm.at[idxv[b].at[r]], buf[b].at[r],
                             msem[b])

    def drain_m(b):
        for r in range(M_CHUNK):
            pltpu.make_async_copy(feats_hbm.at[idxv[b].at[r]], buf[b].at[r],
                                  msem[b]).wait()

    def reduce_chunk(b):
        for r in range(M_CHUNK):
            accs = [buf[b][r, 0, pl.ds(16 * j, 16)] for j in range(D // 16)]

            def acc_body(k, a):
                return tuple(a[j] + buf[b][r, k, pl.ds(16 * j, 16)]
                             for j in range(D // 16))

            accs = lax.fori_loop(1, FAN0, acc_body, tuple(accs))
            for j in range(D // 16):
                obuf[r, pl.ds(16 * j, 16)] = accs[j] * (1.0 / FAN0)

    fire_m(0, 0)
    fire_m(1, 1)

    def pair_body(g, carry):
        for b in range(2):
            c = 2 * g + b
            drain_m(b)
            reduce_chunk(b)
            pltpu.sync_copy(obuf,
                            out_ref.at[pl.ds(out_row0 + c * M_CHUNK, M_CHUNK)])

            @pl.when(c + 2 < n_chunks)
            def _():
                fire_m(c + 2, b)
        return carry

    lax.fori_loop(0, n_chunks // 2, pair_body, 0)


def _sc_body(feats_hbm, n1_hbm, n2n_hbm, idxh_hbm, m1_out, m2_out, hp2_out,
             idxv0, idxv1, buf0, buf1, obuf, idxh, bufh, msem0, msem1, hsem):
    wid = lax.axis_index("s") * NC + lax.axis_index("c")
    h_row0 = wid * HR_PER_W

    pltpu.sync_copy(idxh_hbm.at[pl.ds(h_row0, HR_PER_W)], idxh)
    for j in range(H_GATHERS):
        pltpu.async_copy(feats_hbm.at[idxh.at[pl.ds(j * H_GLEN, H_GLEN)]],
                         bufh.at[pl.ds(j * H_GLEN, H_GLEN)], hsem)

    n1_chunks = N1_CHUNKS // NW
    _mean_pipeline(feats_hbm, n1_hbm, m1_out, (idxv0, idxv1), (buf0, buf1),
                   obuf, (msem0, msem1),
                   chunk0=wid * n1_chunks, n_chunks=n1_chunks,
                   out_row0=wid * (B // NW))

    n2_chunks = (H_ROWS // M_CHUNK) // NW
    _mean_pipeline(feats_hbm, n2n_hbm, m2_out, (idxv0, idxv1), (buf0, buf1),
                   obuf, (msem0, msem1),
                   chunk0=wid * n2_chunks, n_chunks=n2_chunks,
                   out_row0=wid * (H_ROWS // NW))

    for j in range(H_GATHERS):
        pltpu.make_async_copy(feats_hbm.at[idxh.at[pl.ds(j * H_GLEN, H_GLEN)]],
                              bufh.at[pl.ds(j * H_GLEN, H_GLEN)], hsem).wait()
    pltpu.sync_copy(bufh, hp2_out.at[pl.ds(h_row0, HR_PER_W)])


_MEAN_SCRATCH = [
    pltpu.VMEM((M_CHUNK, FAN0), jnp.int32),
    pltpu.VMEM((M_CHUNK, FAN0), jnp.int32),
    pltpu.VMEM((M_CHUNK, FAN0, D), jnp.float32),
    pltpu.VMEM((M_CHUNK, FAN0, D), jnp.float32),
    pltpu.VMEM((M_CHUNK, D), jnp.float32),
]


@jax.jit
def _sc_gather(feats, n1, n2n, idx_h):
    mesh = plsc.VectorSubcoreMesh(core_axis_name="c", subcore_axis_name="s",
                                  num_cores=NC, num_subcores=NS)
    return pl.kernel(
        _sc_body,
        out_type=(
            jax.ShapeDtypeStruct((B, D), jnp.float32),
            jax.ShapeDtypeStruct((B * FAN1, D), jnp.float32),
            jax.ShapeDtypeStruct((H_ROWS, D), jnp.float32),
        ),
        mesh=mesh,
        scratch_types=_MEAN_SCRATCH + [
            pltpu.VMEM((HR_PER_W,), jnp.int32),
            pltpu.VMEM((HR_PER_W, D), jnp.float32),
            pltpu.SemaphoreType.DMA,
            pltpu.SemaphoreType.DMA,
            pltpu.SemaphoreType.DMA,
        ],
    )(feats, n1, n2n, idx_h)


TC_GRID = 8
TC_BLK = H_ROWS // TC_GRID
TC_BB = B // TC_GRID


def _tc_dense_body(x_ref, m1_ref, hp2_ref, m2_ref, w0_ref, b0_ref,
                   w1_ref, b1_ref, out_ref, h0_s, na_s):
    def lin(a, b, w_ref, bias):
        return jnp.maximum(
            jnp.dot(a, w_ref[:D], preferred_element_type=jnp.float32)
            + jnp.dot(b, w_ref[D:], preferred_element_type=jnp.float32)
            + bias, 0.0)

    i = pl.program_id(0)

    @pl.when(i == 0)
    def _():
        h0_s[...] = lin(x_ref[...], m1_ref[...], w0_ref, b0_ref[...])

    h2 = lin(hp2_ref[...], m2_ref[...], w0_ref, b0_ref[...])
    na_s[pl.ds(i * TC_BB, TC_BB), :] = jnp.mean(
        h2.reshape(TC_BB, FAN1, HID), axis=1)

    @pl.when(i == TC_GRID - 1)
    def _():
        out_ref[...] = lin(h0_s[...], na_s[...], w1_ref, b1_ref[...])


@jax.jit
def _tc_dense(x, m1, hp2, m2, W0, b0, W1, b1):
    full = lambda shape: pl.BlockSpec(shape, lambda i: (0, 0))
    return pl.pallas_call(
        _tc_dense_body,
        grid=(TC_GRID,),
        in_specs=[
            full((B, D)),
            full((B, D)),
            pl.BlockSpec((TC_BLK, D), lambda i: (i, 0)),
            pl.BlockSpec((TC_BLK, D), lambda i: (i, 0)),
            full((2 * D, HID)),
            full((1, HID)),
            full((2 * HID, HID)),
            full((1, HID)),
        ],
        out_specs=full((B, HID)),
        out_shape=jax.ShapeDtypeStruct((B, HID), jnp.float32),
        scratch_shapes=[
            pltpu.VMEM((B, HID), jnp.float32),
            pltpu.VMEM((B, HID), jnp.float32),
        ],
    )(x, m1, hp2, m2, W0, b0.reshape(1, HID), W1, b1.reshape(1, HID))


def kernel(x, nodes, feats, neighs1, neighs2, neighs2_neighs, W0, b0, W1, b1):
    idx_h = neighs2.reshape(-1).astype(jnp.int32)
    m1, m2, hp2 = _sc_gather(feats, neighs1.astype(jnp.int32),
                             neighs2_neighs.astype(jnp.int32), idx_h)
    out = _tc_dense(x, m1, hp2, m2, W0, b0, W1, b1)
    return out.reshape(B, 1, HID)


# speedup vs baseline: 1.0198x; 1.0129x over previous
import jax
import jax.numpy as jnp
from jax import lax
from jax.experimental import pallas as pl
from jax.experimental.pallas import tpu as pltpu
from jax.experimental.pallas import tpu_sc as plsc

B = 1024
FAN0 = 25
FAN1 = 10
D = 128
HID = 128

NC = 2
NS = 16
NW = NC * NS

M_ROWS = B + B * FAN1
MR_PER_W = M_ROWS // NW
M_CHUNK = 8
M_CHUNKS = MR_PER_W // M_CHUNK
N1_CHUNKS = B // M_CHUNK

H_ROWS = B * FAN1
HR_PER_W = H_ROWS // NW
H_GLEN = 64
H_GATHERS = HR_PER_W // H_GLEN


def _mean_pipeline(feats_hbm, idx_hbm, out_ref, idxv, buf, obuf, msem,
                   chunk0, n_chunks, out_row0):
    def fire_m(c, b):
        pltpu.sync_copy(idx_hbm.at[pl.ds((chunk0 + c) * M_CHUNK, M_CHUNK)],
                        idxv[b])
        for r in range(M_CHUNK):
            pltpu.async_copy(feats_hbm.at[idxv[b].at[r]], buf[b].at[r],
                             msem[b])

    def drain_m(b):
        for r in range(M_CHUNK):
            pltpu.make_async_copy(feats_hbm.at[idxv[b].at[r]], buf[b].at[r],
                                  msem[b]).wait()

    def reduce_chunk(b):
        for r in range(M_CHUNK):
            accs = [buf[b][r, 0, pl.ds(16 * j, 16)] for j in range(D // 16)]

            def acc_body(k, a):
                return tuple(a[j] + buf[b][r, k, pl.ds(16 * j, 16)]
                             for j in range(D // 16))

            accs = lax.fori_loop(1, FAN0, acc_body, tuple(accs))
            for j in range(D // 16):
                obuf[r, pl.ds(16 * j, 16)] = accs[j] * (1.0 / FAN0)

    fire_m(0, 0)
    fire_m(1, 1)

    def pair_body(g, carry):
        for b in range(2):
            c = 2 * g + b
            drain_m(b)
            reduce_chunk(b)
            pltpu.sync_copy(obuf,
                            out_ref.at[pl.ds(out_row0 + c * M_CHUNK, M_CHUNK)])

            @pl.when(c + 2 < n_chunks)
            def _():
                fire_m(c + 2, b)
        return carry

    lax.fori_loop(0, n_chunks // 2, pair_body, 0)


def _sc_body(feats_hbm, n1_hbm, n2n_hbm, idxh_hbm, m1_out, m2_out, hp2_out,
             idxv0, idxv1, buf0, buf1, obuf, idxh, bufh, msem0, msem1, hsem):
    wid = lax.axis_index("s") * NC + lax.axis_index("c")
    h_row0 = wid * HR_PER_W

    pltpu.sync_copy(idxh_hbm.at[pl.ds(h_row0, HR_PER_W)], idxh)
    for j in range(H_GATHERS):
        pltpu.async_copy(feats_hbm.at[idxh.at[pl.ds(j * H_GLEN, H_GLEN)]],
                         bufh.at[pl.ds(j * H_GLEN, H_GLEN)], hsem)

    n1_chunks = N1_CHUNKS // NW
    _mean_pipeline(feats_hbm, n1_hbm, m1_out, (idxv0, idxv1), (buf0, buf1),
                   obuf, (msem0, msem1),
                   chunk0=wid * n1_chunks, n_chunks=n1_chunks,
                   out_row0=wid * (B // NW))

    n2_chunks = (H_ROWS // M_CHUNK) // NW
    _mean_pipeline(feats_hbm, n2n_hbm, m2_out, (idxv0, idxv1), (buf0, buf1),
                   obuf, (msem0, msem1),
                   chunk0=wid * n2_chunks, n_chunks=n2_chunks,
                   out_row0=wid * (H_ROWS // NW))

    for j in range(H_GATHERS):
        pltpu.make_async_copy(feats_hbm.at[idxh.at[pl.ds(j * H_GLEN, H_GLEN)]],
                              bufh.at[pl.ds(j * H_GLEN, H_GLEN)], hsem).wait()
    pltpu.sync_copy(bufh, hp2_out.at[pl.ds(h_row0, HR_PER_W)])


_MEAN_SCRATCH = [
    pltpu.VMEM((M_CHUNK, FAN0), jnp.int32),
    pltpu.VMEM((M_CHUNK, FAN0), jnp.int32),
    pltpu.VMEM((M_CHUNK, FAN0, D), jnp.float32),
    pltpu.VMEM((M_CHUNK, FAN0, D), jnp.float32),
    pltpu.VMEM((M_CHUNK, D), jnp.float32),
]


@jax.jit
def _sc_gather(feats, n1, n2n, idx_h):
    mesh = plsc.VectorSubcoreMesh(core_axis_name="c", subcore_axis_name="s",
                                  num_cores=NC, num_subcores=NS)
    return pl.kernel(
        _sc_body,
        out_type=(
            jax.ShapeDtypeStruct((B, D), jnp.float32),
            jax.ShapeDtypeStruct((B * FAN1, D), jnp.float32),
            jax.ShapeDtypeStruct((H_ROWS, D), jnp.float32),
        ),
        mesh=mesh,
        scratch_types=_MEAN_SCRATCH + [
            pltpu.VMEM((HR_PER_W,), jnp.int32),
            pltpu.VMEM((HR_PER_W, D), jnp.float32),
            pltpu.SemaphoreType.DMA,
            pltpu.SemaphoreType.DMA,
            pltpu.SemaphoreType.DMA,
        ],
    )(feats, n1, n2n, idx_h)


def _tc_dense_body(x_ref, m1_ref, hp2_ref, m2_ref, w0_ref, b0_ref,
                   w1_ref, b1_ref, out_ref):
    def lin(a, b, w_ref, bias):
        return jnp.maximum(
            jnp.dot(a, w_ref[:D], preferred_element_type=jnp.float32)
            + jnp.dot(b, w_ref[D:], preferred_element_type=jnp.float32)
            + bias, 0.0)

    h0 = lin(x_ref[...], m1_ref[...], w0_ref, b0_ref[...])
    h2 = lin(hp2_ref[...], m2_ref[...], w0_ref, b0_ref[...])
    na = jnp.mean(h2.reshape(B, FAN1, HID), axis=1)
    out_ref[...] = lin(h0, na, w1_ref, b1_ref[...])


@jax.jit
def _tc_dense(x, m1, hp2, m2, W0, b0, W1, b1):
    return pl.pallas_call(
        _tc_dense_body,
        out_shape=jax.ShapeDtypeStruct((B, HID), jnp.float32),
    )(x, m1, hp2, m2, W0, b0.reshape(1, HID), W1, b1.reshape(1, HID))


def kernel(x, nodes, feats, neighs1, neighs2, neighs2_neighs, W0, b0, W1, b1):
    idx_h = neighs2.reshape(-1).astype(jnp.int32)
    m1, m2, hp2 = _sc_gather(feats, neighs1.astype(jnp.int32),
                             neighs2_neighs.astype(jnp.int32), idx_h)
    out = _tc_dense(x, m1, hp2, m2, W0, b0, W1, b1)
    return out.reshape(B, 1, HID)


# speedup vs baseline: 1.0411x; 1.0209x over previous
import jax
import jax.numpy as jnp
from jax import lax
from jax.experimental import pallas as pl
from jax.experimental.pallas import tpu as pltpu
from jax.experimental.pallas import tpu_sc as plsc

B = 1024
FAN0 = 25
FAN1 = 10
D = 128
HID = 128

NC = 2
NS = 16
NW = NC * NS

M_ROWS = B + B * FAN1
MR_PER_W = M_ROWS // NW
M_CHUNK = 8
M_CHUNKS = MR_PER_W // M_CHUNK
N1_CHUNKS = B // M_CHUNK

H_ROWS = B * FAN1
HR_PER_W = H_ROWS // NW
H_GLEN = 64
H_GATHERS = HR_PER_W // H_GLEN


def _sc_gather_body(feats_hbm, n1_hbm, n2n_hbm, idxh_hbm,
                    m1_out, m2_out, hp2_out,
                    idxv0, idxv1, buf0, buf1, obuf, idxh, bufh,
                    msem0, msem1, hsem):
    wid = lax.axis_index("s") * NC + lax.axis_index("c")
    idxv = (idxv0, idxv1)
    buf = (buf0, buf1)
    msem = (msem0, msem1)

    m_row0 = wid * MR_PER_W
    h_row0 = wid * HR_PER_W

    pltpu.sync_copy(idxh_hbm.at[pl.ds(h_row0, HR_PER_W)], idxh)
    for j in range(H_GATHERS):
        pltpu.async_copy(feats_hbm.at[idxh.at[pl.ds(j * H_GLEN, H_GLEN)]],
                         bufh.at[pl.ds(j * H_GLEN, H_GLEN)], hsem)

    def fire_m(c, b):
        g = wid * M_CHUNKS + c

        @pl.when(g < N1_CHUNKS)
        def _():
            pltpu.sync_copy(n1_hbm.at[pl.ds(g * M_CHUNK, M_CHUNK)], idxv[b])

        @pl.when(g >= N1_CHUNKS)
        def _():
            pltpu.sync_copy(n2n_hbm.at[pl.ds(g * M_CHUNK - B, M_CHUNK)],
                            idxv[b])

        for r in range(M_CHUNK):
            pltpu.async_copy(feats_hbm.at[idxv[b].at[r]], buf[b].at[r],
                             msem[b])

    def drain_m(b):
        for r in range(M_CHUNK):
            pltpu.make_async_copy(feats_hbm.at[idxv[b].at[r]], buf[b].at[r],
                                  msem[b]).wait()

    fire_m(0, 0)
    fire_m(1, 1)

    def reduce_chunk(b):
        for r in range(M_CHUNK):
            accs = [buf[b][r, 0, pl.ds(16 * j, 16)] for j in range(D // 16)]

            def acc_body(k, a):
                return tuple(a[j] + buf[b][r, k, pl.ds(16 * j, 16)]
                             for j in range(D // 16))

            accs = lax.fori_loop(1, FAN0, acc_body, tuple(accs))
            for j in range(D // 16):
                obuf[r, pl.ds(16 * j, 16)] = accs[j] * (1.0 / FAN0)

    def write_chunk(c):
        row = m_row0 + c * M_CHUNK

        @pl.when(row < B)
        def _():
            pltpu.sync_copy(obuf, m1_out.at[pl.ds(row, M_CHUNK)])

        @pl.when(row >= B)
        def _():
            pltpu.sync_copy(obuf, m2_out.at[pl.ds(row - B, M_CHUNK)])

    def pair_body(g, carry):
        for b in range(2):
            c = 2 * g + b
            drain_m(b)
            reduce_chunk(b)
            write_chunk(c)

            @pl.when(c + 2 < M_CHUNKS)
            def _():
                fire_m(c + 2, b)
        return carry

    lax.fori_loop(0, M_CHUNKS // 2, pair_body, 0)

    for j in range(H_GATHERS):
        pltpu.make_async_copy(feats_hbm.at[idxh.at[pl.ds(j * H_GLEN, H_GLEN)]],
                              bufh.at[pl.ds(j * H_GLEN, H_GLEN)], hsem).wait()
    pltpu.sync_copy(bufh, hp2_out.at[pl.ds(h_row0, HR_PER_W)])


@jax.jit
def _sc_gather(feats, n1, n2n, idx_h):
    mesh = plsc.VectorSubcoreMesh(core_axis_name="c", subcore_axis_name="s",
                                  num_cores=NC, num_subcores=NS)
    return pl.kernel(
        _sc_gather_body,
        out_type=(
            jax.ShapeDtypeStruct((B, D), jnp.float32),
            jax.ShapeDtypeStruct((B * FAN1, D), jnp.float32),
            jax.ShapeDtypeStruct((H_ROWS, D), jnp.float32),
        ),
        mesh=mesh,
        scratch_types=[
            pltpu.VMEM((M_CHUNK, FAN0), jnp.int32),
            pltpu.VMEM((M_CHUNK, FAN0), jnp.int32),
            pltpu.VMEM((M_CHUNK, FAN0, D), jnp.float32),
            pltpu.VMEM((M_CHUNK, FAN0, D), jnp.float32),
            pltpu.VMEM((M_CHUNK, D), jnp.float32),
            pltpu.VMEM((HR_PER_W,), jnp.int32),
            pltpu.VMEM((HR_PER_W, D), jnp.float32),
            pltpu.SemaphoreType.DMA,
            pltpu.SemaphoreType.DMA,
            pltpu.SemaphoreType.DMA,
        ],
    )(feats, n1, n2n, idx_h)


def _tc_dense_body(x_ref, m1_ref, hp2_ref, m2_ref, w0_ref, b0_ref,
                   w1_ref, b1_ref, out_ref):
    def lin(a, b, w_ref, bias):
        return jnp.maximum(
            jnp.dot(a, w_ref[:D], preferred_element_type=jnp.float32)
            + jnp.dot(b, w_ref[D:], preferred_element_type=jnp.float32)
            + bias, 0.0)

    h0 = lin(x_ref[...], m1_ref[...], w0_ref, b0_ref[...])
    h2 = lin(hp2_ref[...], m2_ref[...], w0_ref, b0_ref[...])
    na = jnp.mean(h2.reshape(B, FAN1, HID), axis=1)
    out_ref[...] = lin(h0, na, w1_ref, b1_ref[...])


@jax.jit
def _tc_dense(x, m1, hp2, m2, W0, b0, W1, b1):
    return pl.pallas_call(
        _tc_dense_body,
        out_shape=jax.ShapeDtypeStruct((B, HID), jnp.float32),
    )(x, m1, hp2, m2, W0, b0.reshape(1, HID), W1, b1.reshape(1, HID))


def kernel(x, nodes, feats, neighs1, neighs2, neighs2_neighs, W0, b0, W1, b1):
    idx_h = neighs2.reshape(-1).astype(jnp.int32)
    m1, m2, hp2 = _sc_gather(feats, neighs1.astype(jnp.int32),
                             neighs2_neighs.astype(jnp.int32), idx_h)
    out = _tc_dense(x, m1, hp2, m2, W0, b0, W1, b1)
    return out.reshape(B, 1, HID)
